# baseline (speedup 1.0000x reference)
import functools

import jax
import jax.numpy as jnp
from jax import lax
from jax.experimental import pallas as pl
from jax.experimental.pallas import tpu as pltpu
from jax.experimental.pallas import tpu_sc as plsc

B = 16384
D = 64
T = 384
NC = 2
NS = 16
NW = NC * NS
RPW = B // NW
CH = 128
NCH = RPW // CH

BLK = 512


@functools.cache
def _sc_gather_one():
    mesh = plsc.VectorSubcoreMesh(core_axis_name="c", subcore_axis_name="s")

    @functools.partial(
        pl.kernel,
        mesh=mesh,
        out_type=jax.ShapeDtypeStruct((B, 2 * D), jnp.float32),
        scratch_types=[
            pltpu.VMEM((RPW,), jnp.int32),
            pltpu.VMEM((RPW // 2, D), jnp.float32),
            pltpu.VMEM((RPW // 2, 2 * D), jnp.float32),
            pltpu.SemaphoreType.DMA,
        ],
        compiler_params=pltpu.CompilerParams(use_tc_tiling_on_sc=True,
                                             needs_layout_passes=False),
    )
    def gather_kernel(ids_hbm, tab_hbm, out_hbm, ids_v, rows_v, wide_v, sem):
        wid = lax.axis_index("s") * NC + lax.axis_index("c")
        base = wid * RPW
        HPW = RPW // 2
        pltpu.sync_copy(ids_hbm.at[pl.ds(base, RPW)], ids_v)

        for h in range(2):
            def chunk(g, _):
                r16 = ids_v[pl.ds(h * HPW + g * 16, 16)]
                for j in range(16):
                    pltpu.async_copy(tab_hbm.at[pl.ds(r16[j], 1)],
                                     rows_v.at[pl.ds(g * 16 + j, 1)], sem)
                return 0
            lax.fori_loop(0, HPW // 16, chunk, 0)

            pltpu.make_async_copy(tab_hbm.at[pl.ds(0, HPW)], rows_v,
                                  sem).wait()

            def widen(r, _):
                def col(k, _):
                    wide_v[r, pl.ds(k * 16, 16)] = rows_v[r, pl.ds(k * 16, 16)]
                    return 0
                lax.fori_loop(0, D // 16, col, 0, unroll=4)
                return 0
            lax.fori_loop(0, HPW, widen, 0)

            pltpu.sync_copy(wide_v, out_hbm.at[pl.ds(base + h * HPW, HPW)])

    return gather_kernel


def _tp_body(i_ref, o_ref):
    o_ref[...] = i_ref[...].T


def _tc_transpose(tabT, n_rows):
    cb = 2048
    grid = (n_rows + cb - 1) // cb
    return pl.pallas_call(
        _tp_body,
        grid=(grid,),
        in_specs=[pl.BlockSpec((D, cb), lambda i: (0, i))],
        out_specs=pl.BlockSpec((cb, D), lambda i: (i, 0)),
        out_shape=jax.ShapeDtypeStruct((n_rows, D), jnp.float32),
    )(tabT)


def _mm_body(x_ref, w_ref, b_ref, o_ref):
    o_ref[...] = jnp.dot(x_ref[...], w_ref[...],
                         preferred_element_type=jnp.float32) + b_ref[...]


def _tc_matmul(x, w, b2):
    return pl.pallas_call(
        _mm_body,
        grid=(B // BLK,),
        in_specs=[
            pl.BlockSpec((BLK, T), lambda i: (i, 0)),
            pl.BlockSpec((T, D), lambda i: (0, 0)),
            pl.BlockSpec((1, D), lambda i: (0, 0)),
        ],
        out_specs=pl.BlockSpec((BLK, D), lambda i: (i, 0)),
        out_shape=jax.ShapeDtypeStruct((B, D), jnp.float32),
    )(x, w, b2)


def _dot_body(e_ref, u_ref, c_ref, o_ref):
    u = u_ref[:, :D]
    c = c_ref[:, :D]
    s = jnp.sum(u * (c + e_ref[...]), axis=1, keepdims=True)
    o_ref[...] = 1.0 / (1.0 + jnp.exp(-s))


def _tc_dot(enc, u_rows, c_rows):
    return pl.pallas_call(
        _dot_body,
        grid=(B // BLK,),
        in_specs=[
            pl.BlockSpec((BLK, D), lambda i: (i, 0)),
            pl.BlockSpec((BLK, 2 * D), lambda i: (i, 0)),
            pl.BlockSpec((BLK, 2 * D), lambda i: (i, 0)),
        ],
        out_specs=pl.BlockSpec((BLK, 1), lambda i: (i, 0)),
        out_shape=jax.ShapeDtypeStruct((B, 1), jnp.float32),
    )(enc, u_rows, c_rows)


def kernel(user_ids, content_ids, encoded_text, user_table, item_table,
           proj_W, proj_b):
    uid = user_ids.astype(jnp.int32)
    cid = content_ids.astype(jnp.int32)
    gather = _sc_gather_one()
    utab_rm = _tc_transpose(user_table.T, user_table.shape[0])
    itab_rm = _tc_transpose(item_table.T, item_table.shape[0])
    u_rows = gather(uid, utab_rm)
    c_rows = gather(cid, itab_rm)
    enc = _tc_matmul(encoded_text, proj_W, proj_b.reshape(1, D))
    return _tc_dot(enc, u_rows, c_rows)


# speedup vs baseline: 1.5801x; 1.5801x over previous
import functools

import jax
import jax.numpy as jnp
from jax import lax
from jax.experimental import pallas as pl
from jax.experimental.pallas import tpu as pltpu
from jax.experimental.pallas import tpu_sc as plsc

B = 16384
D = 64
T = 384
NC = 2
NS = 16
NW = NC * NS
RPW = B // NW
CH = 128
NCH = RPW // CH

BLK = 512


@functools.cache
def _sc_gather_one():
    mesh = plsc.VectorSubcoreMesh(core_axis_name="c", subcore_axis_name="s")

    @functools.partial(
        pl.kernel,
        mesh=mesh,
        out_type=jax.ShapeDtypeStruct((B, 2 * D), jnp.float32),
        scratch_types=[
            pltpu.VMEM((RPW,), jnp.int32),
            pltpu.VMEM((RPW // 2, D), jnp.float32),
            pltpu.VMEM((RPW // 2, 2 * D), jnp.float32),
            pltpu.SemaphoreType.DMA,
        ],
        compiler_params=pltpu.CompilerParams(use_tc_tiling_on_sc=True,
                                             needs_layout_passes=False),
    )
    def gather_kernel(ids_hbm, tab_hbm, out_hbm, ids_v, rows_v, wide_v, sem):
        wid = lax.axis_index("s") * NC + lax.axis_index("c")
        base = wid * RPW
        HPW = RPW // 2
        pltpu.sync_copy(ids_hbm.at[pl.ds(base, RPW)], ids_v)

        for h in range(2):
            def chunk(g, _):
                r16 = ids_v[pl.ds(h * HPW + g * 16, 16)]
                for j in range(16):
                    pltpu.async_copy(tab_hbm.at[pl.ds(r16[j], 1)],
                                     rows_v.at[pl.ds(g * 16 + j, 1)], sem)
                return 0
            lax.fori_loop(0, HPW // 16, chunk, 0)

            pltpu.make_async_copy(tab_hbm.at[pl.ds(0, HPW)], rows_v,
                                  sem).wait()

            def widen(r, _):
                def col(k, _):
                    wide_v[r, pl.ds(k * 16, 16)] = rows_v[r, pl.ds(k * 16, 16)]
                    return 0
                lax.fori_loop(0, D // 16, col, 0, unroll=4)
                return 0
            lax.fori_loop(0, HPW, widen, 0)

            pltpu.sync_copy(wide_v, out_hbm.at[pl.ds(base + h * HPW, HPW)])

    return gather_kernel


def _tp_body(i_ref, o_ref):
    o_ref[...] = i_ref[...].T


def _tc_transpose(tabT, n_rows):
    cb = 8192
    grid = (n_rows + cb - 1) // cb
    return pl.pallas_call(
        _tp_body,
        grid=(grid,),
        in_specs=[pl.BlockSpec((D, cb), lambda i: (0, i))],
        out_specs=pl.BlockSpec((cb, D), lambda i: (i, 0)),
        out_shape=jax.ShapeDtypeStruct((n_rows, D), jnp.float32),
    )(tabT)


def _mm_body(x_ref, w_ref, b_ref, o_ref):
    o_ref[...] = jnp.dot(x_ref[...], w_ref[...],
                         preferred_element_type=jnp.float32) + b_ref[...]


def _tc_matmul(x, w, b2):
    return pl.pallas_call(
        _mm_body,
        grid=(B // BLK,),
        in_specs=[
            pl.BlockSpec((BLK, T), lambda i: (i, 0)),
            pl.BlockSpec((T, D), lambda i: (0, 0)),
            pl.BlockSpec((1, D), lambda i: (0, 0)),
        ],
        out_specs=pl.BlockSpec((BLK, D), lambda i: (i, 0)),
        out_shape=jax.ShapeDtypeStruct((B, D), jnp.float32),
    )(x, w, b2)


def _dot_body(e_ref, u_ref, c_ref, o_ref):
    u = u_ref[:, :D]
    c = c_ref[:, :D]
    s = jnp.sum(u * (c + e_ref[...]), axis=1, keepdims=True)
    o_ref[...] = 1.0 / (1.0 + jnp.exp(-s))


def _tc_dot(enc, u_rows, c_rows):
    return pl.pallas_call(
        _dot_body,
        grid=(B // BLK,),
        in_specs=[
            pl.BlockSpec((BLK, D), lambda i: (i, 0)),
            pl.BlockSpec((BLK, 2 * D), lambda i: (i, 0)),
            pl.BlockSpec((BLK, 2 * D), lambda i: (i, 0)),
        ],
        out_specs=pl.BlockSpec((BLK, 1), lambda i: (i, 0)),
        out_shape=jax.ShapeDtypeStruct((B, 1), jnp.float32),
    )(enc, u_rows, c_rows)


def kernel(user_ids, content_ids, encoded_text, user_table, item_table,
           proj_W, proj_b):
    uid = user_ids.astype(jnp.int32)
    cid = content_ids.astype(jnp.int32)
    gather = _sc_gather_one()
    utab_rm = _tc_transpose(user_table.T, user_table.shape[0])
    itab_rm = _tc_transpose(item_table.T, item_table.shape[0])
    u_rows = gather(uid, utab_rm)
    c_rows = gather(cid, itab_rm)
    enc = _tc_matmul(encoded_text, proj_W, proj_b.reshape(1, D))
    return _tc_dot(enc, u_rows, c_rows)


# speedup vs baseline: 1.6710x; 1.0575x over previous
import functools

import jax
import jax.numpy as jnp
from jax import lax
from jax.experimental import pallas as pl
from jax.experimental.pallas import tpu as pltpu
from jax.experimental.pallas import tpu_sc as plsc

B = 16384
D = 64
T = 384
NC = 2
NS = 16
NW = NC * NS
RPW = B // NW
CH = 128
NCH = RPW // CH

BLK = 512


@functools.cache
def _sc_gather_one():
    mesh = plsc.VectorSubcoreMesh(core_axis_name="c", subcore_axis_name="s")

    @functools.partial(
        pl.kernel,
        mesh=mesh,
        out_type=jax.ShapeDtypeStruct((B, 2 * D), jnp.float32),
        scratch_types=[
            pltpu.VMEM((RPW,), jnp.int32),
            pltpu.VMEM((RPW // 2, D), jnp.float32),
            pltpu.VMEM((RPW // 2, 2 * D), jnp.float32),
            pltpu.SemaphoreType.DMA,
        ],
        compiler_params=pltpu.CompilerParams(use_tc_tiling_on_sc=True,
                                             needs_layout_passes=False),
    )
    def gather_kernel(ids_hbm, tab_hbm, out_hbm, ids_v, rows_v, wide_v, sem):
        wid = lax.axis_index("s") * NC + lax.axis_index("c")
        base = wid * RPW
        HPW = RPW // 2
        pltpu.sync_copy(ids_hbm.at[pl.ds(base, RPW)], ids_v)

        for h in range(2):
            def chunk(g, _):
                r16 = ids_v[pl.ds(h * HPW + g * 16, 16)]
                for j in range(16):
                    pltpu.async_copy(tab_hbm.at[pl.ds(r16[j], 1)],
                                     rows_v.at[pl.ds(g * 16 + j, 1)], sem)
                return 0
            lax.fori_loop(0, HPW // 16, chunk, 0)

            pltpu.make_async_copy(tab_hbm.at[pl.ds(0, HPW)], rows_v,
                                  sem).wait()

            def widen(r, _):
                def col(k, _):
                    wide_v[r, pl.ds(k * 16, 16)] = rows_v[r, pl.ds(k * 16, 16)]
                    return 0
                lax.fori_loop(0, D // 16, col, 0, unroll=4)
                return 0
            lax.fori_loop(0, HPW, widen, 0)

            pltpu.sync_copy(wide_v, out_hbm.at[pl.ds(base + h * HPW, HPW)])

    return gather_kernel


def _tp_body(i_ref, o_ref):
    o_ref[...] = i_ref[...].T


def _tc_transpose(tabT, n_rows):
    cb = 16384
    grid = (n_rows + cb - 1) // cb
    return pl.pallas_call(
        _tp_body,
        grid=(grid,),
        in_specs=[pl.BlockSpec((D, cb), lambda i: (0, i))],
        out_specs=pl.BlockSpec((cb, D), lambda i: (i, 0)),
        out_shape=jax.ShapeDtypeStruct((n_rows, D), jnp.float32),
    )(tabT)


def _mm_body(x_ref, w_ref, b_ref, o_ref):
    o_ref[...] = jnp.dot(x_ref[...], w_ref[...],
                         preferred_element_type=jnp.float32) + b_ref[...]


def _tc_matmul(x, w, b2):
    return pl.pallas_call(
        _mm_body,
        grid=(B // BLK,),
        in_specs=[
            pl.BlockSpec((BLK, T), lambda i: (i, 0)),
            pl.BlockSpec((T, D), lambda i: (0, 0)),
            pl.BlockSpec((1, D), lambda i: (0, 0)),
        ],
        out_specs=pl.BlockSpec((BLK, D), lambda i: (i, 0)),
        out_shape=jax.ShapeDtypeStruct((B, D), jnp.float32),
    )(x, w, b2)


def _dot_body(e_ref, u_ref, c_ref, o_ref):
    u = u_ref[:, :D]
    c = c_ref[:, :D]
    s = jnp.sum(u * (c + e_ref[...]), axis=1, keepdims=True)
    o_ref[...] = 1.0 / (1.0 + jnp.exp(-s))


def _tc_dot(enc, u_rows, c_rows):
    return pl.pallas_call(
        _dot_body,
        grid=(B // BLK,),
        in_specs=[
            pl.BlockSpec((BLK, D), lambda i: (i, 0)),
            pl.BlockSpec((BLK, 2 * D), lambda i: (i, 0)),
            pl.BlockSpec((BLK, 2 * D), lambda i: (i, 0)),
        ],
        out_specs=pl.BlockSpec((BLK, 1), lambda i: (i, 0)),
        out_shape=jax.ShapeDtypeStruct((B, 1), jnp.float32),
    )(enc, u_rows, c_rows)


def kernel(user_ids, content_ids, encoded_text, user_table, item_table,
           proj_W, proj_b):
    uid = user_ids.astype(jnp.int32)
    cid = content_ids.astype(jnp.int32)
    gather = _sc_gather_one()
    utab_rm = _tc_transpose(user_table.T, user_table.shape[0])
    itab_rm = _tc_transpose(item_table.T, item_table.shape[0])
    u_rows = gather(uid, utab_rm)
    c_rows = gather(cid, itab_rm)
    enc = _tc_matmul(encoded_text, proj_W, proj_b.reshape(1, D))
    return _tc_dot(enc, u_rows, c_rows)


# speedup vs baseline: 1.7079x; 1.0221x over previous
import functools

import jax
import jax.numpy as jnp
from jax import lax
from jax.experimental import pallas as pl
from jax.experimental.pallas import tpu as pltpu
from jax.experimental.pallas import tpu_sc as plsc

B = 16384
D = 64
T = 384
NC = 2
NS = 16
NW = NC * NS
RPW = B // NW
CH = 128
NCH = RPW // CH

BLK = 512


@functools.cache
def _sc_gather_one():
    mesh = plsc.VectorSubcoreMesh(core_axis_name="c", subcore_axis_name="s")

    @functools.partial(
        pl.kernel,
        mesh=mesh,
        out_type=jax.ShapeDtypeStruct((B, 2 * D), jnp.float32),
        scratch_types=[
            pltpu.VMEM((RPW,), jnp.int32),
            pltpu.VMEM((RPW // 2, D), jnp.float32),
            pltpu.VMEM((RPW // 2, 2 * D), jnp.float32),
            pltpu.SemaphoreType.DMA,
        ],
        compiler_params=pltpu.CompilerParams(use_tc_tiling_on_sc=True,
                                             needs_layout_passes=False),
    )
    def gather_kernel(ids_hbm, tab_hbm, out_hbm, ids_v, rows_v, wide_v, sem):
        wid = lax.axis_index("s") * NC + lax.axis_index("c")
        base = wid * RPW
        HPW = RPW // 2
        pltpu.sync_copy(ids_hbm.at[pl.ds(base, RPW)], ids_v)

        for h in range(2):
            def chunk(g, _):
                r16 = ids_v[pl.ds(h * HPW + g * 16, 16)]
                for j in range(16):
                    pltpu.async_copy(tab_hbm.at[pl.ds(r16[j], 1)],
                                     rows_v.at[pl.ds(g * 16 + j, 1)], sem)
                return 0
            lax.fori_loop(0, HPW // 16, chunk, 0)

            pltpu.make_async_copy(tab_hbm.at[pl.ds(0, HPW)], rows_v,
                                  sem).wait()

            def widen(r, _):
                def col(k, _):
                    wide_v[r, pl.ds(k * 16, 16)] = rows_v[r, pl.ds(k * 16, 16)]
                    return 0
                lax.fori_loop(0, D // 16, col, 0, unroll=4)
                return 0
            lax.fori_loop(0, HPW, widen, 0)

            pltpu.sync_copy(wide_v, out_hbm.at[pl.ds(base + h * HPW, HPW)])

    return gather_kernel


def _tp_body(i_ref, o_ref):
    o_ref[...] = i_ref[...].T


def _tc_transpose(tabT, n_rows):
    cb = 32768
    grid = (n_rows + cb - 1) // cb
    return pl.pallas_call(
        _tp_body,
        grid=(grid,),
        in_specs=[pl.BlockSpec((D, cb), lambda i: (0, i))],
        out_specs=pl.BlockSpec((cb, D), lambda i: (i, 0)),
        out_shape=jax.ShapeDtypeStruct((n_rows, D), jnp.float32),
    )(tabT)


def _mm_body(x_ref, w_ref, b_ref, o_ref):
    o_ref[...] = jnp.dot(x_ref[...], w_ref[...],
                         preferred_element_type=jnp.float32) + b_ref[...]


def _tc_matmul(x, w, b2):
    return pl.pallas_call(
        _mm_body,
        grid=(B // BLK,),
        in_specs=[
            pl.BlockSpec((BLK, T), lambda i: (i, 0)),
            pl.BlockSpec((T, D), lambda i: (0, 0)),
            pl.BlockSpec((1, D), lambda i: (0, 0)),
        ],
        out_specs=pl.BlockSpec((BLK, D), lambda i: (i, 0)),
        out_shape=jax.ShapeDtypeStruct((B, D), jnp.float32),
    )(x, w, b2)


def _dot_body(e_ref, u_ref, c_ref, o_ref):
    u = u_ref[:, :D]
    c = c_ref[:, :D]
    s = jnp.sum(u * (c + e_ref[...]), axis=1, keepdims=True)
    o_ref[...] = 1.0 / (1.0 + jnp.exp(-s))


def _tc_dot(enc, u_rows, c_rows):
    return pl.pallas_call(
        _dot_body,
        grid=(B // BLK,),
        in_specs=[
            pl.BlockSpec((BLK, D), lambda i: (i, 0)),
            pl.BlockSpec((BLK, 2 * D), lambda i: (i, 0)),
            pl.BlockSpec((BLK, 2 * D), lambda i: (i, 0)),
        ],
        out_specs=pl.BlockSpec((BLK, 1), lambda i: (i, 0)),
        out_shape=jax.ShapeDtypeStruct((B, 1), jnp.float32),
    )(enc, u_rows, c_rows)


def kernel(user_ids, content_ids, encoded_text, user_table, item_table,
           proj_W, proj_b):
    uid = user_ids.astype(jnp.int32)
    cid = content_ids.astype(jnp.int32)
    gather = _sc_gather_one()
    utab_rm = _tc_transpose(user_table.T, user_table.shape[0])
    itab_rm = _tc_transpose(item_table.T, item_table.shape[0])
    u_rows = gather(uid, utab_rm)
    c_rows = gather(cid, itab_rm)
    enc = _tc_matmul(encoded_text, proj_W, proj_b.reshape(1, D))
    return _tc_dot(enc, u_rows, c_rows)


# speedup vs baseline: 1.7774x; 1.0407x over previous
import functools

import jax
import jax.numpy as jnp
from jax import lax
from jax.experimental import pallas as pl
from jax.experimental.pallas import tpu as pltpu
from jax.experimental.pallas import tpu_sc as plsc

B = 16384
D = 64
T = 384
NC = 2
NS = 16
NW = NC * NS
RPW = B // NW
CH = 128
NCH = RPW // CH

BLK = 512


@functools.cache
def _sc_gather_one():
    mesh = plsc.VectorSubcoreMesh(core_axis_name="c", subcore_axis_name="s")

    @functools.partial(
        pl.kernel,
        mesh=mesh,
        out_type=jax.ShapeDtypeStruct((B, 2 * D), jnp.float32),
        scratch_types=[
            pltpu.VMEM((RPW,), jnp.int32),
            pltpu.VMEM((RPW // 2, D), jnp.float32),
            pltpu.VMEM((RPW // 2, 2 * D), jnp.float32),
            pltpu.SemaphoreType.DMA,
        ],
        compiler_params=pltpu.CompilerParams(use_tc_tiling_on_sc=True,
                                             needs_layout_passes=False),
    )
    def gather_kernel(ids_hbm, tab_hbm, out_hbm, ids_v, rows_v, wide_v, sem):
        wid = lax.axis_index("s") * NC + lax.axis_index("c")
        base = wid * RPW
        HPW = RPW // 2
        pltpu.sync_copy(ids_hbm.at[pl.ds(base, RPW)], ids_v)

        for h in range(2):
            def chunk(g, _):
                r16 = ids_v[pl.ds(h * HPW + g * 16, 16)]
                for j in range(16):
                    pltpu.async_copy(tab_hbm.at[pl.ds(r16[j], 1)],
                                     rows_v.at[pl.ds(g * 16 + j, 1)], sem)
                return 0
            lax.fori_loop(0, HPW // 16, chunk, 0)

            pltpu.make_async_copy(tab_hbm.at[pl.ds(0, HPW)], rows_v,
                                  sem).wait()

            def widen(r, _):
                def col(k, _):
                    wide_v[r, pl.ds(k * 16, 16)] = rows_v[r, pl.ds(k * 16, 16)]
                    return 0
                lax.fori_loop(0, D // 16, col, 0, unroll=4)
                return 0
            lax.fori_loop(0, HPW, widen, 0)

            pltpu.sync_copy(wide_v, out_hbm.at[pl.ds(base + h * HPW, HPW)])

    return gather_kernel


def _tp_body(i_ref, o_ref):
    o_ref[...] = i_ref[...].T


def _tc_transpose(tabT, n_rows):
    cb = 32768
    grid = (n_rows + cb - 1) // cb
    return pl.pallas_call(
        _tp_body,
        grid=(grid,),
        in_specs=[pl.BlockSpec((D, cb), lambda i: (0, i))],
        out_specs=pl.BlockSpec((cb, D), lambda i: (i, 0)),
        out_shape=jax.ShapeDtypeStruct((n_rows, D), jnp.float32),
    )(tabT)


def _dot_body(x_ref, w_ref, b_ref, u_ref, c_ref, o_ref):
    enc = jnp.dot(x_ref[...], w_ref[...],
                  preferred_element_type=jnp.float32) + b_ref[...]
    u = u_ref[:, :D]
    c = c_ref[:, :D]
    s = jnp.sum(u * (c + enc), axis=1, keepdims=True)
    o_ref[...] = 1.0 / (1.0 + jnp.exp(-s))


def _tc_dot(x, w, b2, u_rows, c_rows):
    return pl.pallas_call(
        _dot_body,
        grid=(B // BLK,),
        in_specs=[
            pl.BlockSpec((BLK, T), lambda i: (i, 0)),
            pl.BlockSpec((T, D), lambda i: (0, 0)),
            pl.BlockSpec((1, D), lambda i: (0, 0)),
            pl.BlockSpec((BLK, 2 * D), lambda i: (i, 0)),
            pl.BlockSpec((BLK, 2 * D), lambda i: (i, 0)),
        ],
        out_specs=pl.BlockSpec((BLK, 1), lambda i: (i, 0)),
        out_shape=jax.ShapeDtypeStruct((B, 1), jnp.float32),
    )(x, w, b2, u_rows, c_rows)


def kernel(user_ids, content_ids, encoded_text, user_table, item_table,
           proj_W, proj_b):
    uid = user_ids.astype(jnp.int32)
    cid = content_ids.astype(jnp.int32)
    gather = _sc_gather_one()
    utab_rm = _tc_transpose(user_table.T, user_table.shape[0])
    itab_rm = _tc_transpose(item_table.T, item_table.shape[0])
    u_rows = gather(uid, utab_rm)
    c_rows = gather(cid, itab_rm)
    return _tc_dot(encoded_text, proj_W, proj_b.reshape(1, D),
                   u_rows, c_rows)


# speedup vs baseline: 1.7792x; 1.0010x over previous
import functools

import jax
import jax.numpy as jnp
from jax import lax
from jax.experimental import pallas as pl
from jax.experimental.pallas import tpu as pltpu
from jax.experimental.pallas import tpu_sc as plsc

B = 16384
D = 64
T = 384
NC = 2
NS = 16
NW = NC * NS
RPW = B // NW
CH = 128
NCH = RPW // CH

BLK = 512


@functools.cache
def _sc_gather_one():
    mesh = plsc.VectorSubcoreMesh(core_axis_name="c", subcore_axis_name="s")

    @functools.partial(
        pl.kernel,
        mesh=mesh,
        out_type=jax.ShapeDtypeStruct((B, 2 * D), jnp.float32),
        scratch_types=[
            pltpu.VMEM((RPW,), jnp.int32),
            pltpu.VMEM((RPW // 2, D), jnp.float32),
            pltpu.VMEM((RPW // 2, 2 * D), jnp.float32),
            pltpu.SemaphoreType.DMA,
        ],
        compiler_params=pltpu.CompilerParams(use_tc_tiling_on_sc=True,
                                             needs_layout_passes=False),
    )
    def gather_kernel(ids_hbm, tab_hbm, out_hbm, ids_v, rows_v, wide_v, sem):
        wid = lax.axis_index("s") * NC + lax.axis_index("c")
        base = wid * RPW
        HPW = RPW // 2
        pltpu.sync_copy(ids_hbm.at[pl.ds(base, RPW)], ids_v)

        for h in range(2):
            def chunk(g, _):
                r16 = ids_v[pl.ds(h * HPW + g * 16, 16)]
                for j in range(16):
                    pltpu.async_copy(tab_hbm.at[pl.ds(r16[j], 1)],
                                     rows_v.at[pl.ds(g * 16 + j, 1)], sem)
                return 0
            lax.fori_loop(0, HPW // 16, chunk, 0)

            pltpu.make_async_copy(tab_hbm.at[pl.ds(0, HPW)], rows_v,
                                  sem).wait()

            def widen(r, _):
                def col(k, _):
                    wide_v[r, pl.ds(k * 16, 16)] = rows_v[r, pl.ds(k * 16, 16)]
                    return 0
                lax.fori_loop(0, D // 16, col, 0, unroll=4)
                return 0
            lax.fori_loop(0, HPW, widen, 0)

            pltpu.sync_copy(wide_v, out_hbm.at[pl.ds(base + h * HPW, HPW)])

    return gather_kernel


def _tp_body(i_ref, o_ref):
    o_ref[...] = i_ref[...].T


def _tc_transpose(tabT, n_rows):
    cb = 32768
    grid = (n_rows + cb - 1) // cb
    return pl.pallas_call(
        _tp_body,
        grid=(grid,),
        in_specs=[pl.BlockSpec((D, cb), lambda i: (0, i))],
        out_specs=pl.BlockSpec((cb, D), lambda i: (i, 0)),
        out_shape=jax.ShapeDtypeStruct((n_rows, D), jnp.float32),
    )(tabT)


def _dot_body(x_ref, w_ref, b_ref, u_ref, c_ref, o_ref):
    enc = jnp.dot(x_ref[...], w_ref[...],
                  preferred_element_type=jnp.float32) + b_ref[...]
    u = u_ref[:, :D]
    c = c_ref[:, :D]
    s = jnp.sum(u * (c + enc), axis=1, keepdims=True)
    o_ref[...] = 1.0 / (1.0 + jnp.exp(-s))


def _tc_dot(x, w, b2, u_rows, c_rows):
    return pl.pallas_call(
        _dot_body,
        grid=(B // BLK,),
        in_specs=[
            pl.BlockSpec((BLK, T), lambda i: (i, 0)),
            pl.BlockSpec((T, D), lambda i: (0, 0)),
            pl.BlockSpec((1, D), lambda i: (0, 0)),
            pl.BlockSpec((BLK, 2 * D), lambda i: (i, 0)),
            pl.BlockSpec((BLK, 2 * D), lambda i: (i, 0)),
        ],
        out_specs=pl.BlockSpec((BLK, 1), lambda i: (i, 0)),
        out_shape=jax.ShapeDtypeStruct((B, 1), jnp.float32),
    )(x, w, b2, u_rows, c_rows)


def kernel(user_ids, content_ids, encoded_text, user_table, item_table,
           proj_W, proj_b):
    uid = user_ids.astype(jnp.int32)
    cid = content_ids.astype(jnp.int32)
    gather = _sc_gather_one()
    itab_rm = _tc_transpose(item_table.T, item_table.shape[0])
    c_rows = gather(cid, itab_rm)
    utab_rm = _tc_transpose(user_table.T, user_table.shape[0])
    u_rows = gather(uid, utab_rm)
    return _tc_dot(encoded_text, proj_W, proj_b.reshape(1, D),
                   u_rows, c_rows)
